# baseline (speedup 1.0000x reference)
import functools

import jax
import jax.numpy as jnp
from jax.experimental import pallas as pl
from jax.experimental.pallas import tpu as pltpu

D_MODEL_C = 64
D_SAE_C = 16384
K_C = 32
BATCH_C = 4096
N_DEAD_THRESH = 10.0


def _encode_body(x_ref, w_ref, benc_ref, bdec_ref, acts_ref):
    xc = x_ref[...] - bdec_ref[...][None, :]
    pre = jnp.dot(xc, w_ref[...], preferred_element_type=jnp.float32)
    acts_ref[...] = jnp.maximum(pre + benc_ref[...][None, :], 0.0)


def _encode(x, W_enc, b_enc, b_dec):
    B, D = x.shape
    S = W_enc.shape[1]
    BLK = 512
    grid = (S // BLK,)
    return pl.pallas_call(
        _encode_body,
        grid=grid,
        in_specs=[
            pl.BlockSpec((B, D), lambda j: (0, 0)),
            pl.BlockSpec((D, BLK), lambda j: (0, j)),
            pl.BlockSpec((BLK,), lambda j: (j,)),
            pl.BlockSpec((D,), lambda j: (0,)),
        ],
        out_specs=pl.BlockSpec((B, BLK), lambda j: (0, j)),
        out_shape=jax.ShapeDtypeStruct((B, S), jnp.float32),
    )(x, W_enc, b_enc, b_dec)


def _decode_body(a_ref, w_ref, x_ref, bdec_ref, rec_ref, sums_ref):
    j = pl.program_id(0)
    a = a_ref[...]
    part = jnp.dot(a, w_ref[...], preferred_element_type=jnp.float32)

    @pl.when(j == 0)
    def _init():
        rec_ref[...] = jnp.zeros_like(rec_ref)
        sums_ref[...] = jnp.zeros_like(sums_ref)

    rec_ref[...] += part
    lane = jax.lax.broadcasted_iota(jnp.int32, (1, 8), 1)
    s_l1 = jnp.sum(jnp.abs(a))
    s_l0 = jnp.sum((a > 0.0).astype(jnp.float32))
    sums_ref[...] += jnp.where(lane == 0, s_l1, 0.0) + jnp.where(lane == 1, s_l0, 0.0)

    @pl.when(j == pl.num_programs(0) - 1)
    def _fin():
        rec = rec_ref[...] + bdec_ref[...][None, :]
        rec_ref[...] = rec
        d = rec - x_ref[...]
        s_l2 = jnp.sum(d * d)
        sums_ref[...] += jnp.where(lane == 2, s_l2, 0.0)


def _decode(acts_topk, W_dec, x, b_dec):
    B, S = acts_topk.shape
    D = W_dec.shape[1]
    BLK = 512
    rec, sums = pl.pallas_call(
        _decode_body,
        grid=(S // BLK,),
        in_specs=[
            pl.BlockSpec((B, BLK), lambda j: (0, j)),
            pl.BlockSpec((BLK, D), lambda j: (j, 0)),
            pl.BlockSpec((B, D), lambda j: (0, 0)),
            pl.BlockSpec((D,), lambda j: (0,)),
        ],
        out_specs=[
            pl.BlockSpec((B, D), lambda j: (0, 0)),
            pl.BlockSpec((1, 8), lambda j: (0, 0)),
        ],
        out_shape=[
            jax.ShapeDtypeStruct((B, D), jnp.float32),
            jax.ShapeDtypeStruct((1, 8), jnp.float32),
        ],
    )(acts_topk, W_dec, x, b_dec)
    return rec, sums


def kernel(x, W_enc, W_dec, b_enc, b_dec, num_batches_not_active):
    B = x.shape[0]
    total_k = K_C * B
    acts = _encode(x, W_enc, b_enc, b_dec)
    acts_flat = acts.reshape(-1)
    topk_vals, topk_idx = jax.lax.top_k(acts_flat, total_k)
    acts_topk = jnp.zeros_like(acts_flat).at[topk_idx].set(topk_vals).reshape(acts.shape)
    x_reconstruct, sums = _decode(acts_topk, W_dec, x, b_dec)
    l1_norm = sums[0, 0] / B
    l0_norm = sums[0, 1] / B
    l2_loss = sums[0, 2] / (B * x.shape[1])
    l1_loss = 0.0 * l1_norm
    aux_loss = jnp.array(0.0, dtype=x.dtype)
    loss = l2_loss + l1_loss + aux_loss
    num_dead_features = jnp.sum((num_batches_not_active > N_DEAD_THRESH).astype(jnp.int32))
    return (x_reconstruct, acts_topk, loss, l2_loss, l1_loss, l0_norm,
            l1_norm, aux_loss, num_dead_features)


# speedup vs baseline: 36.3474x; 36.3474x over previous
import functools

import jax
import jax.numpy as jnp
from jax import lax
from jax.experimental import pallas as pl
from jax.experimental.pallas import tpu as pltpu
from jax.experimental.pallas import tpu_sc as plsc

K_C = 32
N_DEAD_THRESH = 10.0

NC = 2
NS = 16
NW = NC * NS

NB_A = 32768
NB_B = 65536
TOTAL = 4096 * 16384
PER_W = TOTAL // NW
CH = 8192
N_CHUNKS = PER_W // CH


def _encode_body(x_ref, w_ref, benc_ref, bdec_ref, bits_ref):
    xc = x_ref[...] - bdec_ref[...][None, :]
    pre = jnp.dot(xc, w_ref[...], preferred_element_type=jnp.float32)
    acts = jnp.maximum(pre + benc_ref[...][None, :], 0.0)
    bits_ref[...] = lax.bitcast_convert_type(acts, jnp.int32)


def _encode_bits(x, W_enc, b_enc, b_dec):
    B, D = x.shape
    S = W_enc.shape[1]
    BLK = 512
    return pl.pallas_call(
        _encode_body,
        grid=(S // BLK,),
        in_specs=[
            pl.BlockSpec((B, D), lambda j: (0, 0)),
            pl.BlockSpec((D, BLK), lambda j: (0, j)),
            pl.BlockSpec((BLK,), lambda j: (j,)),
            pl.BlockSpec((D,), lambda j: (0,)),
        ],
        out_specs=pl.BlockSpec((B, BLK), lambda j: (0, j)),
        out_shape=jax.ShapeDtypeStruct((B, S), jnp.int32),
    )(x, W_enc, b_enc, b_dec)


def _hist_zero(hist_ref, nbins):
    def zbody(i, c):
        hist_ref[pl.ds(i * 16, 16)] = jnp.zeros((16,), jnp.int32)
        return c
    lax.fori_loop(0, nbins // 16, zbody, 0)


def _make_pass_a():
    mesh = plsc.VectorSubcoreMesh(
        core_axis_name="c", subcore_axis_name="s",
        num_cores=NC, num_subcores=NS)

    @functools.partial(
        pl.kernel,
        out_type=jax.ShapeDtypeStruct((NW, NB_A), jnp.int32),
        mesh=mesh,
        scratch_types=[
            pltpu.VMEM((CH,), jnp.int32),
            pltpu.VMEM((NB_A,), jnp.int32),
        ],
        compiler_params=pltpu.CompilerParams(needs_layout_passes=False),
    )
    def k(bits_hbm, hist_out, chunk_v, hist_v):
        wid = lax.axis_index("s") * NC + lax.axis_index("c")
        _hist_zero(hist_v, NB_A)
        ones = jnp.ones((16,), jnp.int32)

        def chunk_body(i, c):
            base = wid * PER_W + i * CH
            pltpu.sync_copy(bits_hbm.at[pl.ds(base, CH)], chunk_v)

            def vbody(k2, c2):
                v = chunk_v[pl.ds(k2 * 16, 16)]
                pos = v > 0
                hi = lax.shift_right_logical(v, 16)
                plsc.addupdate_scatter(hist_v, [hi], ones, mask=pos)
                return c2
            lax.fori_loop(0, CH // 16, vbody, 0, unroll=8)
            return c
        lax.fori_loop(0, N_CHUNKS, chunk_body, 0)
        pltpu.sync_copy(hist_v, hist_out.at[wid])

    return k


def _make_pass_b():
    mesh = plsc.VectorSubcoreMesh(
        core_axis_name="c", subcore_axis_name="s",
        num_cores=NC, num_subcores=NS)

    @functools.partial(
        pl.kernel,
        out_type=jax.ShapeDtypeStruct((NW, NB_B), jnp.int32),
        mesh=mesh,
        scratch_types=[
            pltpu.VMEM((CH,), jnp.int32),
            pltpu.VMEM((NB_B,), jnp.int32),
            pltpu.VMEM((16,), jnp.int32),
        ],
        compiler_params=pltpu.CompilerParams(needs_layout_passes=False),
    )
    def k(bits_hbm, b1_hbm, hist_out, chunk_v, hist_v, b1_v):
        wid = lax.axis_index("s") * NC + lax.axis_index("c")
        _hist_zero(hist_v, NB_B)
        pltpu.sync_copy(b1_hbm, b1_v)
        b1 = b1_v[...]
        ones = jnp.ones((16,), jnp.int32)
        lomask = jnp.full((16,), 0xFFFF, jnp.int32)

        def chunk_body(i, c):
            base = wid * PER_W + i * CH
            pltpu.sync_copy(bits_hbm.at[pl.ds(base, CH)], chunk_v)

            def vbody(k2, c2):
                v = chunk_v[pl.ds(k2 * 16, 16)]
                hi = lax.shift_right_logical(v, 16)
                m = (v > 0) & (hi == b1)
                lo = v & lomask
                plsc.addupdate_scatter(hist_v, [lo], ones, mask=m)
                return c2
            lax.fori_loop(0, CH // 16, vbody, 0, unroll=8)
            return c
        lax.fori_loop(0, N_CHUNKS, chunk_body, 0)
        pltpu.sync_copy(hist_v, hist_out.at[wid])

    return k


_PASS_A = _make_pass_a()
_PASS_B = _make_pass_b()


def _suffix_sum(h):
    return jnp.cumsum(h[::-1])[::-1]


def _finish_body(tau_ref, x_ref, wenc_ref, benc_ref, bdec_ref, wdec_ref,
                 topk_ref, rec_ref, sums_ref):
    j = pl.program_id(0)
    tau = tau_ref[0]
    xc = x_ref[...] - bdec_ref[...][None, :]
    pre = jnp.dot(xc, wenc_ref[...], preferred_element_type=jnp.float32)
    acts = jnp.maximum(pre + benc_ref[...][None, :], 0.0)
    a = jnp.where(acts >= tau, acts, 0.0)
    topk_ref[...] = a
    part = jnp.dot(a, wdec_ref[...], preferred_element_type=jnp.float32)

    @pl.when(j == 0)
    def _init():
        rec_ref[...] = jnp.zeros_like(rec_ref)
        sums_ref[...] = jnp.zeros_like(sums_ref)

    rec_ref[...] += part
    lane = lax.broadcasted_iota(jnp.int32, (1, 8), 1)
    s_l1 = jnp.sum(a)
    s_l0 = jnp.sum((a > 0.0).astype(jnp.float32))
    sums_ref[...] += jnp.where(lane == 0, s_l1, 0.0) + jnp.where(lane == 1, s_l0, 0.0)

    @pl.when(j == pl.num_programs(0) - 1)
    def _fin():
        rec = rec_ref[...] + bdec_ref[...][None, :]
        rec_ref[...] = rec
        d = rec - x_ref[...]
        s_l2 = jnp.sum(d * d)
        sums_ref[...] += jnp.where(lane == 2, s_l2, 0.0)


def _finish(tau, x, W_enc, b_enc, b_dec, W_dec):
    B, D = x.shape
    S = W_enc.shape[1]
    BLK = 512
    topk, rec, sums = pl.pallas_call(
        _finish_body,
        grid=(S // BLK,),
        in_specs=[
            pl.BlockSpec(memory_space=pltpu.SMEM),
            pl.BlockSpec((B, D), lambda j: (0, 0)),
            pl.BlockSpec((D, BLK), lambda j: (0, j)),
            pl.BlockSpec((BLK,), lambda j: (j,)),
            pl.BlockSpec((D,), lambda j: (0,)),
            pl.BlockSpec((BLK, D), lambda j: (j, 0)),
        ],
        out_specs=[
            pl.BlockSpec((B, BLK), lambda j: (0, j)),
            pl.BlockSpec((B, D), lambda j: (0, 0)),
            pl.BlockSpec((1, 8), lambda j: (0, 0)),
        ],
        out_shape=[
            jax.ShapeDtypeStruct((B, S), jnp.float32),
            jax.ShapeDtypeStruct((B, D), jnp.float32),
            jax.ShapeDtypeStruct((1, 8), jnp.float32),
        ],
    )(tau, x, W_enc, b_enc, b_dec, W_dec)
    return topk, rec, sums


def kernel(x, W_enc, W_dec, b_enc, b_dec, num_batches_not_active):
    B, D = x.shape
    total_k = K_C * B

    bits2d = _encode_bits(x, W_enc, b_enc, b_dec)
    bits = bits2d.reshape(-1)

    hist_a = _PASS_A(bits)
    h_a = jnp.sum(hist_a, axis=0)
    s_a = _suffix_sum(h_a)
    iota_a = jnp.arange(NB_A, dtype=jnp.int32)
    b1 = jnp.max(jnp.where(s_a >= total_k, iota_a, -1))
    s_a_pad = jnp.concatenate([s_a, jnp.zeros((1,), s_a.dtype)])
    c_above = s_a_pad[jnp.clip(b1 + 1, 0, NB_A)]
    r = total_k - c_above

    b1_vec = jnp.full((16,), jnp.maximum(b1, 0), jnp.int32)
    hist_b = _PASS_B(bits, b1_vec)
    h_b = jnp.sum(hist_b, axis=0)
    s_b = _suffix_sum(h_b)
    iota_b = jnp.arange(NB_B, dtype=jnp.int32)
    t2 = jnp.maximum(jnp.max(jnp.where(s_b >= r, iota_b, -1)), 0)

    tau_bits = jnp.where(b1 < 0, 0, (jnp.maximum(b1, 0) << 16) | t2)
    tau = lax.bitcast_convert_type(tau_bits.astype(jnp.int32), jnp.float32)
    tau_arr = tau.reshape(1)

    acts_topk, x_reconstruct, sums = _finish(tau_arr, x, W_enc, b_enc, b_dec, W_dec)

    l1_norm = sums[0, 0] / B
    l0_norm = sums[0, 1] / B
    l2_loss = sums[0, 2] / (B * D)
    l1_loss = 0.0 * l1_norm
    aux_loss = jnp.array(0.0, dtype=x.dtype)
    loss = l2_loss + l1_loss + aux_loss
    num_dead_features = jnp.sum((num_batches_not_active > N_DEAD_THRESH).astype(jnp.int32))
    return (x_reconstruct, acts_topk, loss, l2_loss, l1_loss, l0_norm,
            l1_norm, aux_loss, num_dead_features)
